# Initial kernel scaffold; baseline (speedup 1.0000x reference)
import jax
import jax.numpy as jnp
from jax.experimental import pallas as pl


def kernel(x, mask, face):
    raise NotImplementedError("write your pallas kernel here")


# speedup vs baseline: 6.3505x; 6.3505x over previous
import functools

import jax
import jax.numpy as jnp
from jax import lax
from jax.experimental import pallas as pl
from jax.experimental.pallas import tpu as pltpu
from jax.experimental.pallas import tpu_sc as plsc

_C = 128


@functools.lru_cache(maxsize=None)
def _gather_mean_kernel(n_rows: int, d: int, n_pad: int):
    info = plsc.get_sparse_core_info()
    nc, ns = info.num_cores, info.num_subcores
    nw = nc * ns
    per_w = n_pad // nw
    n_chunks = per_w // _C

    mesh = plsc.VectorSubcoreMesh(core_axis_name="c", subcore_axis_name="s")

    @functools.partial(
        pl.kernel,
        mesh=mesh,
        out_type=jax.ShapeDtypeStruct((n_pad, d), jnp.float32),
        scratch_types=[
            pltpu.VMEM((_C,), jnp.int32),
            pltpu.VMEM((_C,), jnp.int32),
            pltpu.VMEM((_C,), jnp.int32),
            pltpu.VMEM((_C, d), jnp.float32),
            pltpu.VMEM((_C, d), jnp.float32),
            pltpu.VMEM((_C, d), jnp.float32),
            pltpu.VMEM((_C, d), jnp.float32),
            pltpu.SemaphoreType.DMA,
        ],
    )
    def body(xf_hbm, i0_hbm, i1_hbm, i2_hbm, out_hbm,
             i0v, i1v, i2v, b0, b1, b2, ov, sem):
        wid = lax.axis_index("s") * nc + lax.axis_index("c")
        first_chunk = wid * n_chunks

        def chunk(k, carry):
            crow = first_chunk + k
            start = crow * _C
            pltpu.sync_copy(i0_hbm.at[crow], i0v)
            pltpu.sync_copy(i1_hbm.at[crow], i1v)
            pltpu.sync_copy(i2_hbm.at[crow], i2v)
            c0 = pltpu.async_copy(xf_hbm.at[i0v], b0, sem)
            c1 = pltpu.async_copy(xf_hbm.at[i1v], b1, sem)
            c2 = pltpu.async_copy(xf_hbm.at[i2v], b2, sem)
            c0.wait()
            c1.wait()
            c2.wait()

            def row(r, _):
                for dd in range(d // 16):
                    s = pl.ds(dd * 16, 16)
                    ov[r, s] = (b0[r, s] + b1[r, s] + b2[r, s]) / 3.0
                return 0

            lax.fori_loop(0, _C, row, 0)
            pltpu.sync_copy(ov, out_hbm.at[pl.ds(start, _C)])
            return carry

        lax.fori_loop(0, n_chunks, chunk, 0)

    return body, nw


def kernel(x, mask, face):
    B, V, D = x.shape
    F = face.shape[1]

    pf = jnp.where(mask[:, :, None] == 1, face, 0)
    idx = pf + (jnp.arange(B, dtype=pf.dtype) * V)[:, None, None]
    idx_flat = idx.reshape(B * F, 3)

    tile_quant = 32 * _C
    n_pad = ((B * F + tile_quant - 1) // tile_quant) * tile_quant
    pad = n_pad - B * F
    idx_pad = jnp.pad(idx_flat, ((0, pad), (0, 0))).T
    i0 = idx_pad[0].reshape(n_pad // _C, _C)
    i1 = idx_pad[1].reshape(n_pad // _C, _C)
    i2 = idx_pad[2].reshape(n_pad // _C, _C)

    body, _ = _gather_mean_kernel(B * V, D, n_pad)
    out_flat = body(x.reshape(B * V, D), i0, i1, i2)
    add_feat = out_flat[: B * F].reshape(B, F, D)

    new_verts = jnp.concatenate([x, add_feat], axis=1)

    n0 = jnp.broadcast_to(
        (jnp.arange(F, dtype=pf.dtype) + V)[None, :], (B, F)
    )
    v0, v1, v2 = pf[..., 0], pf[..., 1], pf[..., 2]
    nf0 = jnp.stack([n0, v0, v1], axis=2)
    nf1 = jnp.stack([n0, v1, v2], axis=2)
    nf2 = jnp.stack([n0, v2, v0], axis=2)
    new_faces = jnp.concatenate([nf0, nf1, nf2], axis=1)

    row0 = jnp.concatenate([n0, n0, n0, v0, v1, v2, v1, v2, v0], axis=1)
    row1 = jnp.concatenate([v0, v1, v2, v1, v2, v0, n0, n0, n0], axis=1)
    new_edges = jnp.stack([row0, row1], axis=1)

    return (new_verts, new_faces, new_edges)


# speedup vs baseline: 7.1273x; 1.1223x over previous
import functools

import jax
import jax.numpy as jnp
from jax import lax
from jax.experimental import pallas as pl
from jax.experimental.pallas import tpu as pltpu
from jax.experimental.pallas import tpu_sc as plsc

_C = 64
_THIRD = 0.3333333432674408


@functools.lru_cache(maxsize=None)
def _gather_mean_kernel(n_rows: int, d: int, n_pad: int):
    info = plsc.get_sparse_core_info()
    nc, ns = info.num_cores, info.num_subcores
    nw = nc * ns
    per_w = n_pad // nw
    n_chunks = per_w // _C
    assert n_chunks % 2 == 0

    mesh = plsc.VectorSubcoreMesh(core_axis_name="c", subcore_axis_name="s")

    @functools.partial(
        pl.kernel,
        mesh=mesh,
        out_type=jax.ShapeDtypeStruct((n_pad, d), jnp.float32),
        scratch_types=[
            pltpu.VMEM((2, _C), jnp.int32),
            pltpu.VMEM((2, _C), jnp.int32),
            pltpu.VMEM((2, _C), jnp.int32),
            pltpu.VMEM((2, _C, d), jnp.float32),
            pltpu.VMEM((2, _C, d), jnp.float32),
            pltpu.VMEM((2, _C, d), jnp.float32),
            pltpu.VMEM((2, _C, d), jnp.float32),
            pltpu.SemaphoreType.DMA,
            pltpu.SemaphoreType.DMA,
            pltpu.SemaphoreType.DMA,
            pltpu.SemaphoreType.DMA,
        ],
    )
    def body(xf_hbm, i0_hbm, i1_hbm, i2_hbm, out_hbm,
             i0v, i1v, i2v, b0, b1, b2, ov, gsem0, gsem1, ssem0, ssem1):
        wid = lax.axis_index("s") * nc + lax.axis_index("c")
        first_chunk = wid * n_chunks
        gsems = (gsem0, gsem1)
        ssems = (ssem0, ssem1)

        def stage(k, s):
            crow = first_chunk + k
            pltpu.sync_copy(i0_hbm.at[crow], i0v.at[s])
            pltpu.sync_copy(i1_hbm.at[crow], i1v.at[s])
            pltpu.sync_copy(i2_hbm.at[crow], i2v.at[s])
            pltpu.async_copy(xf_hbm.at[i0v.at[s]], b0.at[s], gsems[s])
            pltpu.async_copy(xf_hbm.at[i1v.at[s]], b1.at[s], gsems[s])
            pltpu.async_copy(xf_hbm.at[i2v.at[s]], b2.at[s], gsems[s])

        def drain_gathers(s):
            for buf in (b0, b1, b2):
                pltpu.make_async_copy(
                    xf_hbm.at[i0v.at[s]], buf.at[s], gsems[s]
                ).wait()

        def drain_store(s):
            pltpu.make_async_copy(
                ov.at[s], out_hbm.at[pl.ds(0, _C)], ssems[s]
            ).wait()

        stage(0, 0)

        def outer(k2, carry):
            for s in range(2):
                k = k2 * 2 + s
                nxt = 1 - s

                @pl.when(k + 1 < n_chunks)
                def _():
                    stage(k + 1, nxt)

                drain_gathers(s)

                def row(r, _):
                    for dd in range(d // 16):
                        sl = pl.ds(dd * 16, 16)
                        ov[s, r, sl] = (
                            b0[s, r, sl] + b1[s, r, sl] + b2[s, r, sl]
                        ) * _THIRD
                    return 0

                lax.fori_loop(0, _C, row, 0)

                @pl.when(k >= 2)
                def _():
                    drain_store(s)

                start = (first_chunk + k) * _C
                pltpu.async_copy(
                    ov.at[s], out_hbm.at[pl.ds(start, _C)], ssems[s]
                )
            return carry

        lax.fori_loop(0, n_chunks // 2, outer, 0)
        drain_store(0)
        drain_store(1)

    return body


def kernel(x, mask, face):
    B, V, D = x.shape
    F = face.shape[1]

    pf = jnp.where(mask[:, :, None] == 1, face, 0)
    idx = pf + (jnp.arange(B, dtype=pf.dtype) * V)[:, None, None]
    idx_flat = idx.reshape(B * F, 3)

    tile_quant = 32 * 2 * _C
    n_pad = ((B * F + tile_quant - 1) // tile_quant) * tile_quant
    pad = n_pad - B * F
    idx_pad = jnp.pad(idx_flat, ((0, pad), (0, 0))).T
    i0 = idx_pad[0].reshape(n_pad // _C, _C)
    i1 = idx_pad[1].reshape(n_pad // _C, _C)
    i2 = idx_pad[2].reshape(n_pad // _C, _C)

    body = _gather_mean_kernel(B * V, D, n_pad)
    out_flat = body(x.reshape(B * V, D), i0, i1, i2)
    add_feat = out_flat[: B * F].reshape(B, F, D)

    new_verts = jnp.concatenate([x, add_feat], axis=1)

    n0 = jnp.broadcast_to(
        (jnp.arange(F, dtype=pf.dtype) + V)[None, :], (B, F)
    )
    v0, v1, v2 = pf[..., 0], pf[..., 1], pf[..., 2]
    nf0 = jnp.stack([n0, v0, v1], axis=2)
    nf1 = jnp.stack([n0, v1, v2], axis=2)
    nf2 = jnp.stack([n0, v2, v0], axis=2)
    new_faces = jnp.concatenate([nf0, nf1, nf2], axis=1)

    row0 = jnp.concatenate([n0, n0, n0, v0, v1, v2, v1, v2, v0], axis=1)
    row1 = jnp.concatenate([v0, v1, v2, v1, v2, v0, n0, n0, n0], axis=1)
    new_edges = jnp.stack([row0, row1], axis=1)

    return (new_verts, new_faces, new_edges)
